# Initial kernel scaffold; baseline (speedup 1.0000x reference)
import jax
import jax.numpy as jnp
from jax.experimental import pallas as pl


def kernel(x, edge_index, batch, W11, b11, W12, b12, W21, b21, W22, b22, W31, b31, W32, b32, W41, b41, W42, b42):
    raise NotImplementedError("write your pallas kernel here")


# speedup vs baseline: 8.4293x; 8.4293x over previous
import functools

import jax
import jax.numpy as jnp
from jax import lax
from jax.experimental import pallas as pl
from jax.experimental.pallas import tpu as pltpu
from jax.experimental.pallas import tpu_sc as plsc

F32 = jnp.float32

_NC = 2
_NS = 16
_NW = _NC * _NS

_K = 80


def _agg_sc(v, src2, dst2, n_nodes, h):
    cpw = src2.shape[1]
    rb = (n_nodes // _NS) // 8 * 8
    tail = n_nodes - rb * _NS

    mesh = plsc.VectorSubcoreMesh(core_axis_name="c", subcore_axis_name="s")

    @functools.partial(
        pl.kernel,
        mesh=mesh,
        out_type=jax.ShapeDtypeStruct((_NC, n_nodes, h), F32),
        scratch_types=[
            pltpu.VMEM((cpw, _K), jnp.int32),
            pltpu.VMEM((cpw, _K), jnp.int32),
            pltpu.VMEM((_K, h), F32),
            pltpu.VMEM_SHARED((n_nodes, h), F32),
            pltpu.SemaphoreType.DMA,
        ],
        compiler_params=pltpu.CompilerParams(use_tc_tiling_on_sc=False),
    )
    def body(v_hbm, src_hbm, dst_hbm, out_hbm, src_v, dst_v, rows_v,
             acc, sem):
        c = lax.axis_index("c")
        s = lax.axis_index("s")
        wid = s * _NC + c

        pltpu.sync_copy(v_hbm.at[pl.ds(s * rb, rb)], acc.at[pl.ds(s * rb, rb)])

        @pl.when(s == _NS - 1)
        def _():
            pltpu.sync_copy(v_hbm.at[pl.ds(rb * _NS, tail)],
                            acc.at[pl.ds(rb * _NS, tail)])

        plsc.subcore_barrier()

        pltpu.sync_copy(src_hbm.at[wid], src_v)
        pltpu.sync_copy(dst_hbm.at[wid], dst_v)

        def chunk(j, carry):
            pltpu.async_copy(v_hbm.at[src_v.at[j]], rows_v, sem).wait()
            pltpu.sync_copy(rows_v, acc.at[dst_v.at[j]], add=True)
            return carry

        lax.fori_loop(0, cpw, chunk, 0)
        plsc.subcore_barrier()

        pltpu.sync_copy(acc.at[pl.ds(s * rb, rb)],
                        out_hbm.at[c, pl.ds(s * rb, rb)])

        @pl.when(s == _NS - 1)
        def _():
            pltpu.sync_copy(acc.at[pl.ds(rb * _NS, tail)],
                            out_hbm.at[c, pl.ds(rb * _NS, tail)])

    return body(v, src2, dst2)


def _row_specs(br, dims):
    return [pl.BlockSpec((br, d), lambda i: (0, 0) if fixed else (i, 0))
            for d, fixed in dims]


def _full_spec(shape):
    nd = len(shape)
    return pl.BlockSpec(shape, lambda i: (0,) * nd)


def _blk_spec(br, d):
    return pl.BlockSpec((br, d), lambda i: (i, 0))


def _mm1_body(x_ref, w_ref, o_ref):
    o_ref[...] = jnp.dot(x_ref[...], w_ref[...], preferred_element_type=F32)


def _mm1(x, w, br):
    n = x.shape[0]
    return pl.pallas_call(
        _mm1_body,
        grid=(n // br,),
        in_specs=[_blk_spec(br, x.shape[1]), _full_spec(w.shape)],
        out_specs=_blk_spec(br, w.shape[1]),
        out_shape=jax.ShapeDtypeStruct((n, w.shape[1]), F32),
    )(x, w)


def _post_body(v_ref, a0_ref, a1_ref, b1_ref, w2_ref, b2_ref, wn_ref, o_ref):
    z = jnp.maximum(a0_ref[...] + a1_ref[...] - v_ref[...] + b1_ref[...], 0.0)
    xo = jnp.dot(z, w2_ref[...], preferred_element_type=F32) + b2_ref[...]
    o_ref[...] = jnp.dot(xo, wn_ref[...], preferred_element_type=F32)


def _post(v, a0, a1, b1, w2, b2, wnext, br):
    n, h = v.shape
    return pl.pallas_call(
        _post_body,
        grid=(n // br,),
        in_specs=[_blk_spec(br, h), _blk_spec(br, h), _blk_spec(br, h),
                  _full_spec((1, h)), _full_spec(w2.shape),
                  _full_spec((1, w2.shape[1])), _full_spec(wnext.shape)],
        out_specs=_blk_spec(br, wnext.shape[1]),
        out_shape=jax.ShapeDtypeStruct((n, wnext.shape[1]), F32),
    )(v, a0, a1, b1.reshape(1, -1), w2, b2.reshape(1, -1), wnext)


def _post2_body(v_ref, a0_ref, a1_ref, b1_ref, w2_ref, b2_ref,
                wa_ref, side_ref, wb_ref, x2_ref, v_next_ref):
    z = jnp.maximum(a0_ref[...] + a1_ref[...] - v_ref[...] + b1_ref[...], 0.0)
    xo = jnp.dot(z, w2_ref[...], preferred_element_type=F32) + b2_ref[...]
    x2_ref[...] = xo
    v_next_ref[...] = (jnp.dot(side_ref[...], wa_ref[...],
                               preferred_element_type=F32)
                       + jnp.dot(xo, wb_ref[...], preferred_element_type=F32))


def _post2(v, a0, a1, b1, w2, b2, wa, side, wb, br):
    n, h = v.shape
    ho = w2.shape[1]
    hn = wb.shape[1]
    return pl.pallas_call(
        _post2_body,
        grid=(n // br,),
        in_specs=[_blk_spec(br, h), _blk_spec(br, h), _blk_spec(br, h),
                  _full_spec((1, h)), _full_spec(w2.shape),
                  _full_spec((1, ho)), _full_spec(wa.shape),
                  _blk_spec(br, side.shape[1]), _full_spec(wb.shape)],
        out_specs=(_blk_spec(br, ho), _blk_spec(br, hn)),
        out_shape=(jax.ShapeDtypeStruct((n, ho), F32),
                   jax.ShapeDtypeStruct((n, hn), F32)),
    )(v, a0, a1, b1.reshape(1, -1), w2, b2.reshape(1, -1), wa, side, wb)


def _pool_body(v_ref, a0_ref, a1_ref, b1_ref, w2_ref, b2_ref, bat_ref,
               o_ref, cnt_ref, *, g, br, nblk):
    i = pl.program_id(0)

    @pl.when(i == 0)
    def _():
        o_ref[...] = jnp.zeros_like(o_ref)
        cnt_ref[...] = jnp.zeros_like(cnt_ref)

    z = jnp.maximum(a0_ref[...] + a1_ref[...] - v_ref[...] + b1_ref[...], 0.0)
    xo = jnp.dot(z, w2_ref[...], preferred_element_type=F32) + b2_ref[...]
    c = xo.shape[1]
    gid = lax.broadcasted_iota(jnp.int32, (g, br), 0).astype(F32)
    oh = jnp.where(jnp.broadcast_to(bat_ref[0], (g, br)) == gid, 1.0, 0.0)
    o_ref[...] += jnp.dot(oh, xo, preferred_element_type=F32)
    cnt_ref[...] += jnp.dot(oh, jnp.ones((br, c), F32),
                            preferred_element_type=F32)

    @pl.when(i == nblk - 1)
    def _():
        o_ref[...] = o_ref[...] / jnp.maximum(cnt_ref[...], 1.0)


def _pool(v, a0, a1, b1, w2, b2, batchf, g, br):
    n, h = v.shape
    c = w2.shape[1]
    nblk = n // br
    return pl.pallas_call(
        functools.partial(_pool_body, g=g, br=br, nblk=nblk),
        grid=(nblk,),
        in_specs=[_blk_spec(br, h), _blk_spec(br, h), _blk_spec(br, h),
                  _full_spec((1, h)), _full_spec(w2.shape),
                  _full_spec((1, c)),
                  pl.BlockSpec((1, 1, br), lambda i: (i, 0, 0))],
        out_specs=pl.BlockSpec((g, c), lambda i: (0, 0)),
        out_shape=jax.ShapeDtypeStruct((g, c), F32),
        scratch_shapes=[pltpu.VMEM((g, c), F32)],
    )(v, a0, a1, b1.reshape(1, -1), w2, b2.reshape(1, -1), batchf)


def kernel(x, edge_index, batch,
           W11, b11, W12, b12,
           W21, b21, W22, b22,
           W31, b31, W32, b32,
           W41, b41, W42, b42):
    n, d = x.shape
    h = W11.shape[1]
    e = edge_index.shape[1]
    g = 64
    c_out = W42.shape[1]
    br = 2000

    src2 = edge_index[0].reshape(_NW, e // (_NW * _K), _K)
    dst2 = edge_index[1].reshape(_NW, e // (_NW * _K), _K)
    batchf = batch.astype(F32).reshape(n // br, 1, br)

    W31a, W31b = W31[:d], W31[d:]
    W41a, W41b = W41[:h], W41[h:]

    v1 = _mm1(x, W11, br)
    a1 = _agg_sc(v1, src2, dst2, n, h)
    v2 = _post(v1, a1[0], a1[1], b11, W12, b12, W21, br)
    a2 = _agg_sc(v2, src2, dst2, n, h)
    x2, v3 = _post2(v2, a2[0], a2[1], b21, W22, b22, W31a, x, W31b, br)
    a3 = _agg_sc(v3, src2, dst2, n, h)
    _, v4 = _post2(v3, a3[0], a3[1], b31, W32, b32, W41b, x2, W41a, br)
    a4 = _agg_sc(v4, src2, dst2, n, h)
    return _pool(v4, a4[0], a4[1], b41, W42, b42, batchf, g, br)


# speedup vs baseline: 12.7769x; 1.5158x over previous
import functools

import jax
import jax.numpy as jnp
from jax import lax
from jax.experimental import pallas as pl
from jax.experimental.pallas import tpu as pltpu
from jax.experimental.pallas import tpu_sc as plsc

F32 = jnp.float32

_NC = 2
_NS = 16
_NW = _NC * _NS

_K = 80


def _agg_sc(v, src2, dst2, n_nodes, h):
    cpw = src2.shape[1]
    rb = (n_nodes // _NS) // 8 * 8
    tail = n_nodes - rb * _NS

    mesh = plsc.VectorSubcoreMesh(core_axis_name="c", subcore_axis_name="s")

    @functools.partial(
        pl.kernel,
        mesh=mesh,
        out_type=jax.ShapeDtypeStruct((_NC, n_nodes, h), F32),
        scratch_types=[
            pltpu.VMEM((cpw, _K), jnp.int32),
            pltpu.VMEM((cpw, _K), jnp.int32),
            pltpu.VMEM((_K, h), F32),
            pltpu.VMEM((_K, h), F32),
            pltpu.VMEM_SHARED((n_nodes, h), F32),
            pltpu.SemaphoreType.DMA,
            pltpu.SemaphoreType.DMA,
        ],
        compiler_params=pltpu.CompilerParams(use_tc_tiling_on_sc=False),
    )
    def body(v_hbm, src_hbm, dst_hbm, out_hbm, src_v, dst_v, rows0, rows1,
             acc, sem0, sem1):
        c = lax.axis_index("c")
        s = lax.axis_index("s")
        wid = s * _NC + c

        pltpu.sync_copy(v_hbm.at[pl.ds(s * rb, rb)], acc.at[pl.ds(s * rb, rb)])

        @pl.when(s == _NS - 1)
        def _():
            pltpu.sync_copy(v_hbm.at[pl.ds(rb * _NS, tail)],
                            acc.at[pl.ds(rb * _NS, tail)])

        plsc.subcore_barrier()

        pltpu.sync_copy(src_hbm.at[wid], src_v)
        pltpu.sync_copy(dst_hbm.at[wid], dst_v)

        pltpu.async_copy(v_hbm.at[src_v.at[0]], rows0, sem0)

        def pipe(i, carry):
            pltpu.async_copy(v_hbm.at[src_v.at[2 * i + 1]], rows1, sem1)
            pltpu.make_async_copy(v_hbm.at[src_v.at[0]], rows0, sem0).wait()
            pltpu.sync_copy(rows0, acc.at[dst_v.at[2 * i]], add=True)
            pltpu.async_copy(v_hbm.at[src_v.at[2 * i + 2]], rows0, sem0)
            pltpu.make_async_copy(v_hbm.at[src_v.at[0]], rows1, sem1).wait()
            pltpu.sync_copy(rows1, acc.at[dst_v.at[2 * i + 1]], add=True)
            return carry

        assert cpw % 2 == 1 and cpw >= 3
        lax.fori_loop(0, (cpw - 1) // 2, pipe, 0)
        pltpu.make_async_copy(v_hbm.at[src_v.at[0]], rows0, sem0).wait()
        pltpu.sync_copy(rows0, acc.at[dst_v.at[cpw - 1]], add=True)
        plsc.subcore_barrier()

        pltpu.sync_copy(acc.at[pl.ds(s * rb, rb)],
                        out_hbm.at[c, pl.ds(s * rb, rb)])

        @pl.when(s == _NS - 1)
        def _():
            pltpu.sync_copy(acc.at[pl.ds(rb * _NS, tail)],
                            out_hbm.at[c, pl.ds(rb * _NS, tail)])

    return body(v, src2, dst2)


def _row_specs(br, dims):
    return [pl.BlockSpec((br, d), lambda i: (0, 0) if fixed else (i, 0))
            for d, fixed in dims]


def _full_spec(shape):
    nd = len(shape)
    return pl.BlockSpec(shape, lambda i: (0,) * nd)


def _blk_spec(br, d):
    return pl.BlockSpec((br, d), lambda i: (i, 0))


def _mm1_body(x_ref, w_ref, o_ref):
    o_ref[...] = jnp.dot(x_ref[...], w_ref[...], preferred_element_type=F32)


def _mm1(x, w, br):
    n = x.shape[0]
    return pl.pallas_call(
        _mm1_body,
        grid=(n // br,),
        in_specs=[_blk_spec(br, x.shape[1]), _full_spec(w.shape)],
        out_specs=_blk_spec(br, w.shape[1]),
        out_shape=jax.ShapeDtypeStruct((n, w.shape[1]), F32),
    )(x, w)


def _post_body(v_ref, a0_ref, a1_ref, b1_ref, w2_ref, b2_ref, wn_ref, o_ref):
    z = jnp.maximum(a0_ref[...] + a1_ref[...] - v_ref[...] + b1_ref[...], 0.0)
    xo = jnp.dot(z, w2_ref[...], preferred_element_type=F32) + b2_ref[...]
    o_ref[...] = jnp.dot(xo, wn_ref[...], preferred_element_type=F32)


def _post(v, a0, a1, b1, w2, b2, wnext, br):
    n, h = v.shape
    return pl.pallas_call(
        _post_body,
        grid=(n // br,),
        in_specs=[_blk_spec(br, h), _blk_spec(br, h), _blk_spec(br, h),
                  _full_spec((1, h)), _full_spec(w2.shape),
                  _full_spec((1, w2.shape[1])), _full_spec(wnext.shape)],
        out_specs=_blk_spec(br, wnext.shape[1]),
        out_shape=jax.ShapeDtypeStruct((n, wnext.shape[1]), F32),
    )(v, a0, a1, b1.reshape(1, -1), w2, b2.reshape(1, -1), wnext)


def _post2_body(v_ref, a0_ref, a1_ref, b1_ref, w2_ref, b2_ref,
                wa_ref, side_ref, wb_ref, x2_ref, v_next_ref):
    z = jnp.maximum(a0_ref[...] + a1_ref[...] - v_ref[...] + b1_ref[...], 0.0)
    xo = jnp.dot(z, w2_ref[...], preferred_element_type=F32) + b2_ref[...]
    x2_ref[...] = xo
    v_next_ref[...] = (jnp.dot(side_ref[...], wa_ref[...],
                               preferred_element_type=F32)
                       + jnp.dot(xo, wb_ref[...], preferred_element_type=F32))


def _post2(v, a0, a1, b1, w2, b2, wa, side, wb, br):
    n, h = v.shape
    ho = w2.shape[1]
    hn = wb.shape[1]
    return pl.pallas_call(
        _post2_body,
        grid=(n // br,),
        in_specs=[_blk_spec(br, h), _blk_spec(br, h), _blk_spec(br, h),
                  _full_spec((1, h)), _full_spec(w2.shape),
                  _full_spec((1, ho)), _full_spec(wa.shape),
                  _blk_spec(br, side.shape[1]), _full_spec(wb.shape)],
        out_specs=(_blk_spec(br, ho), _blk_spec(br, hn)),
        out_shape=(jax.ShapeDtypeStruct((n, ho), F32),
                   jax.ShapeDtypeStruct((n, hn), F32)),
    )(v, a0, a1, b1.reshape(1, -1), w2, b2.reshape(1, -1), wa, side, wb)


def _pool_body(v_ref, a0_ref, a1_ref, b1_ref, w2_ref, b2_ref, bat_ref,
               o_ref, cnt_ref, *, g, br, nblk):
    i = pl.program_id(0)

    @pl.when(i == 0)
    def _():
        o_ref[...] = jnp.zeros_like(o_ref)
        cnt_ref[...] = jnp.zeros_like(cnt_ref)

    z = jnp.maximum(a0_ref[...] + a1_ref[...] - v_ref[...] + b1_ref[...], 0.0)
    xo = jnp.dot(z, w2_ref[...], preferred_element_type=F32) + b2_ref[...]
    c = xo.shape[1]
    gid = lax.broadcasted_iota(jnp.int32, (g, br), 0).astype(F32)
    oh = jnp.where(jnp.broadcast_to(bat_ref[0], (g, br)) == gid, 1.0, 0.0)
    o_ref[...] += jnp.dot(oh, xo, preferred_element_type=F32)
    cnt_ref[...] += jnp.dot(oh, jnp.ones((br, c), F32),
                            preferred_element_type=F32)

    @pl.when(i == nblk - 1)
    def _():
        o_ref[...] = o_ref[...] / jnp.maximum(cnt_ref[...], 1.0)


def _pool(v, a0, a1, b1, w2, b2, batchf, g, br):
    n, h = v.shape
    c = w2.shape[1]
    nblk = n // br
    return pl.pallas_call(
        functools.partial(_pool_body, g=g, br=br, nblk=nblk),
        grid=(nblk,),
        in_specs=[_blk_spec(br, h), _blk_spec(br, h), _blk_spec(br, h),
                  _full_spec((1, h)), _full_spec(w2.shape),
                  _full_spec((1, c)),
                  pl.BlockSpec((1, 1, br), lambda i: (i, 0, 0))],
        out_specs=pl.BlockSpec((g, c), lambda i: (0, 0)),
        out_shape=jax.ShapeDtypeStruct((g, c), F32),
        scratch_shapes=[pltpu.VMEM((g, c), F32)],
    )(v, a0, a1, b1.reshape(1, -1), w2, b2.reshape(1, -1), batchf)


def kernel(x, edge_index, batch,
           W11, b11, W12, b12,
           W21, b21, W22, b22,
           W31, b31, W32, b32,
           W41, b41, W42, b42):
    n, d = x.shape
    h = W11.shape[1]
    e = edge_index.shape[1]
    g = 64
    c_out = W42.shape[1]
    br = 2000

    src2 = edge_index[0].reshape(_NW, e // (_NW * _K), _K)
    dst2 = edge_index[1].reshape(_NW, e // (_NW * _K), _K)
    batchf = batch.astype(F32).reshape(n // br, 1, br)

    W31a, W31b = W31[:d], W31[d:]
    W41a, W41b = W41[:h], W41[h:]

    v1 = _mm1(x, W11, br)
    a1 = _agg_sc(v1, src2, dst2, n, h)
    v2 = _post(v1, a1[0], a1[1], b11, W12, b12, W21, br)
    a2 = _agg_sc(v2, src2, dst2, n, h)
    x2, v3 = _post2(v2, a2[0], a2[1], b21, W22, b22, W31a, x, W31b, br)
    a3 = _agg_sc(v3, src2, dst2, n, h)
    _, v4 = _post2(v3, a3[0], a3[1], b31, W32, b32, W41b, x2, W41a, br)
    a4 = _agg_sc(v4, src2, dst2, n, h)
    return _pool(v4, a4[0], a4[1], b41, W42, b42, batchf, g, br)


# speedup vs baseline: 15.4367x; 1.2082x over previous
import functools

import jax
import jax.numpy as jnp
from jax import lax
from jax.experimental import pallas as pl
from jax.experimental.pallas import tpu as pltpu
from jax.experimental.pallas import tpu_sc as plsc

F32 = jnp.float32

_NC = 2
_NS = 16
_NW = _NC * _NS

_K = 80


def _agg_sc(v, src2, dst2, n_nodes, h):
    cpw = src2.shape[1]
    rb = (n_nodes // _NS) // 8 * 8
    tail = n_nodes - rb * _NS

    mesh = plsc.VectorSubcoreMesh(core_axis_name="c", subcore_axis_name="s")

    @functools.partial(
        pl.kernel,
        mesh=mesh,
        out_type=jax.ShapeDtypeStruct((_NC, n_nodes, h), F32),
        scratch_types=(
            [pltpu.VMEM((cpw, _K), jnp.int32),
             pltpu.VMEM((cpw, _K), jnp.int32)]
            + [pltpu.VMEM((_K, h), F32) for _ in range(8)]
            + [pltpu.VMEM_SHARED((n_nodes, h), F32)]
            + [pltpu.SemaphoreType.DMA for _ in range(4)]
        ),
        compiler_params=pltpu.CompilerParams(use_tc_tiling_on_sc=False),
    )
    def body(v_hbm, src_hbm, dst_hbm, out_hbm, src_v, dst_v, *rest):
        bufs, acc, sems = rest[:8], rest[8], rest[9:]
        buf_a, buf_b = bufs[:4], bufs[4:]
        sga, ssa, sgb, ssb = sems
        c = lax.axis_index("c")
        s = lax.axis_index("s")
        wid = s * _NC + c

        pltpu.sync_copy(v_hbm.at[pl.ds(s * rb, rb)], acc.at[pl.ds(s * rb, rb)])

        @pl.when(s == _NS - 1)
        def _():
            pltpu.sync_copy(v_hbm.at[pl.ds(rb * _NS, tail)],
                            acc.at[pl.ds(rb * _NS, tail)])

        plsc.subcore_barrier()

        pltpu.sync_copy(src_hbm.at[wid], src_v)
        pltpu.sync_copy(dst_hbm.at[wid], dst_v)

        def gath(j, buf, sem):
            pltpu.async_copy(v_hbm.at[src_v.at[j]], buf, sem)

        def gwait(buf, sem):
            pltpu.make_async_copy(v_hbm.at[src_v.at[0]], buf, sem).wait()

        def scat(j, buf, sem):
            pltpu.async_copy(buf, acc.at[dst_v.at[j]], sem, add=True)

        def swait(buf, sem):
            pltpu.make_async_copy(buf, acc.at[dst_v.at[0]], sem).wait()

        nouter = cpw // 8
        assert nouter >= 1
        for b in range(4):
            gath(b, buf_a[b], sga)

        def pipe(i, carry):
            @pl.when(i > 0)
            def _():
                for b in range(4):
                    swait(buf_b[b], ssb)

            for b in range(4):
                gath(8 * i + 4 + b, buf_b[b], sgb)
            for b in range(4):
                gwait(buf_a[b], sga)
            for b in range(4):
                scat(8 * i + b, buf_a[b], ssa)
            @pl.when(i < nouter - 1)
            def _():
                for b in range(4):
                    swait(buf_a[b], ssa)
                for b in range(4):
                    gath(8 * (i + 1) + b, buf_a[b], sga)

            for b in range(4):
                gwait(buf_b[b], sgb)
            for b in range(4):
                scat(8 * i + 4 + b, buf_b[b], ssb)
            return carry

        lax.fori_loop(0, nouter, pipe, 0)
        for b in range(4):
            swait(buf_a[b], ssa)
        for b in range(4):
            swait(buf_b[b], ssb)

        for r in range(8 * nouter, cpw):
            gath(r, buf_a[0], sga)
            gwait(buf_a[0], sga)
            scat(r, buf_a[0], ssa)
            swait(buf_a[0], ssa)
        plsc.subcore_barrier()

        pltpu.sync_copy(acc.at[pl.ds(s * rb, rb)],
                        out_hbm.at[c, pl.ds(s * rb, rb)])

        @pl.when(s == _NS - 1)
        def _():
            pltpu.sync_copy(acc.at[pl.ds(rb * _NS, tail)],
                            out_hbm.at[c, pl.ds(rb * _NS, tail)])

    return body(v, src2, dst2)


def _row_specs(br, dims):
    return [pl.BlockSpec((br, d), lambda i: (0, 0) if fixed else (i, 0))
            for d, fixed in dims]


def _full_spec(shape):
    nd = len(shape)
    return pl.BlockSpec(shape, lambda i: (0,) * nd)


def _blk_spec(br, d):
    return pl.BlockSpec((br, d), lambda i: (i, 0))


def _mm1_body(x_ref, w_ref, o_ref):
    o_ref[...] = jnp.dot(x_ref[...], w_ref[...], preferred_element_type=F32)


def _mm1(x, w, br):
    n = x.shape[0]
    return pl.pallas_call(
        _mm1_body,
        grid=(n // br,),
        in_specs=[_blk_spec(br, x.shape[1]), _full_spec(w.shape)],
        out_specs=_blk_spec(br, w.shape[1]),
        out_shape=jax.ShapeDtypeStruct((n, w.shape[1]), F32),
    )(x, w)


def _post_body(v_ref, a0_ref, a1_ref, b1_ref, w2_ref, b2_ref, wn_ref, o_ref):
    z = jnp.maximum(a0_ref[...] + a1_ref[...] - v_ref[...] + b1_ref[...], 0.0)
    xo = jnp.dot(z, w2_ref[...], preferred_element_type=F32) + b2_ref[...]
    o_ref[...] = jnp.dot(xo, wn_ref[...], preferred_element_type=F32)


def _post(v, a0, a1, b1, w2, b2, wnext, br):
    n, h = v.shape
    return pl.pallas_call(
        _post_body,
        grid=(n // br,),
        in_specs=[_blk_spec(br, h), _blk_spec(br, h), _blk_spec(br, h),
                  _full_spec((1, h)), _full_spec(w2.shape),
                  _full_spec((1, w2.shape[1])), _full_spec(wnext.shape)],
        out_specs=_blk_spec(br, wnext.shape[1]),
        out_shape=jax.ShapeDtypeStruct((n, wnext.shape[1]), F32),
    )(v, a0, a1, b1.reshape(1, -1), w2, b2.reshape(1, -1), wnext)


def _post2_body(v_ref, a0_ref, a1_ref, b1_ref, w2_ref, b2_ref,
                wa_ref, side_ref, wb_ref, x2_ref, v_next_ref):
    z = jnp.maximum(a0_ref[...] + a1_ref[...] - v_ref[...] + b1_ref[...], 0.0)
    xo = jnp.dot(z, w2_ref[...], preferred_element_type=F32) + b2_ref[...]
    x2_ref[...] = xo
    v_next_ref[...] = (jnp.dot(side_ref[...], wa_ref[...],
                               preferred_element_type=F32)
                       + jnp.dot(xo, wb_ref[...], preferred_element_type=F32))


def _post2(v, a0, a1, b1, w2, b2, wa, side, wb, br):
    n, h = v.shape
    ho = w2.shape[1]
    hn = wb.shape[1]
    return pl.pallas_call(
        _post2_body,
        grid=(n // br,),
        in_specs=[_blk_spec(br, h), _blk_spec(br, h), _blk_spec(br, h),
                  _full_spec((1, h)), _full_spec(w2.shape),
                  _full_spec((1, ho)), _full_spec(wa.shape),
                  _blk_spec(br, side.shape[1]), _full_spec(wb.shape)],
        out_specs=(_blk_spec(br, ho), _blk_spec(br, hn)),
        out_shape=(jax.ShapeDtypeStruct((n, ho), F32),
                   jax.ShapeDtypeStruct((n, hn), F32)),
    )(v, a0, a1, b1.reshape(1, -1), w2, b2.reshape(1, -1), wa, side, wb)


def _pool_body(v_ref, a0_ref, a1_ref, b1_ref, w2_ref, b2_ref, bat_ref,
               o_ref, cnt_ref, *, g, br, nblk):
    i = pl.program_id(0)

    @pl.when(i == 0)
    def _():
        o_ref[...] = jnp.zeros_like(o_ref)
        cnt_ref[...] = jnp.zeros_like(cnt_ref)

    z = jnp.maximum(a0_ref[...] + a1_ref[...] - v_ref[...] + b1_ref[...], 0.0)
    xo = jnp.dot(z, w2_ref[...], preferred_element_type=F32) + b2_ref[...]
    c = xo.shape[1]
    gid = lax.broadcasted_iota(jnp.int32, (g, br), 0).astype(F32)
    oh = jnp.where(jnp.broadcast_to(bat_ref[0], (g, br)) == gid, 1.0, 0.0)
    o_ref[...] += jnp.dot(oh, xo, preferred_element_type=F32)
    cnt_ref[...] += jnp.dot(oh, jnp.ones((br, c), F32),
                            preferred_element_type=F32)

    @pl.when(i == nblk - 1)
    def _():
        o_ref[...] = o_ref[...] / jnp.maximum(cnt_ref[...], 1.0)


def _pool(v, a0, a1, b1, w2, b2, batchf, g, br):
    n, h = v.shape
    c = w2.shape[1]
    nblk = n // br
    return pl.pallas_call(
        functools.partial(_pool_body, g=g, br=br, nblk=nblk),
        grid=(nblk,),
        in_specs=[_blk_spec(br, h), _blk_spec(br, h), _blk_spec(br, h),
                  _full_spec((1, h)), _full_spec(w2.shape),
                  _full_spec((1, c)),
                  pl.BlockSpec((1, 1, br), lambda i: (i, 0, 0))],
        out_specs=pl.BlockSpec((g, c), lambda i: (0, 0)),
        out_shape=jax.ShapeDtypeStruct((g, c), F32),
        scratch_shapes=[pltpu.VMEM((g, c), F32)],
    )(v, a0, a1, b1.reshape(1, -1), w2, b2.reshape(1, -1), batchf)


def kernel(x, edge_index, batch,
           W11, b11, W12, b12,
           W21, b21, W22, b22,
           W31, b31, W32, b32,
           W41, b41, W42, b42):
    n, d = x.shape
    h = W11.shape[1]
    e = edge_index.shape[1]
    g = 64
    c_out = W42.shape[1]
    br = 2000

    src2 = edge_index[0].reshape(_NW, e // (_NW * _K), _K)
    dst2 = edge_index[1].reshape(_NW, e // (_NW * _K), _K)
    batchf = batch.astype(F32).reshape(n // br, 1, br)

    W31a, W31b = W31[:d], W31[d:]
    W41a, W41b = W41[:h], W41[h:]

    v1 = _mm1(x, W11, br)
    a1 = _agg_sc(v1, src2, dst2, n, h)
    v2 = _post(v1, a1[0], a1[1], b11, W12, b12, W21, br)
    a2 = _agg_sc(v2, src2, dst2, n, h)
    x2, v3 = _post2(v2, a2[0], a2[1], b21, W22, b22, W31a, x, W31b, br)
    a3 = _agg_sc(v3, src2, dst2, n, h)
    _, v4 = _post2(v3, a3[0], a3[1], b31, W32, b32, W41b, x2, W41a, br)
    a4 = _agg_sc(v4, src2, dst2, n, h)
    return _pool(v4, a4[0], a4[1], b41, W42, b42, batchf, g, br)


# speedup vs baseline: 16.5957x; 1.0751x over previous
import functools

import jax
import jax.numpy as jnp
from jax import lax
from jax.experimental import pallas as pl
from jax.experimental.pallas import tpu as pltpu
from jax.experimental.pallas import tpu_sc as plsc

F32 = jnp.float32

_NC = 2
_NS = 16
_NW = _NC * _NS

_K = 80


def _agg_sc(v, src2, dst2, n_nodes, h):
    cpw = src2.shape[1]
    rb = (n_nodes // _NS) // 8 * 8
    tail = n_nodes - rb * _NS

    mesh = plsc.VectorSubcoreMesh(core_axis_name="c", subcore_axis_name="s")

    @functools.partial(
        pl.kernel,
        mesh=mesh,
        out_type=(jax.ShapeDtypeStruct((n_nodes, h), F32),
                  jax.ShapeDtypeStruct((n_nodes, h), F32)),
        scratch_types=(
            [pltpu.VMEM((cpw, _K), jnp.int32),
             pltpu.VMEM((cpw, _K), jnp.int32)]
            + [pltpu.VMEM((_K, h), F32) for _ in range(8)]
            + [pltpu.VMEM_SHARED((n_nodes, h), F32)]
            + [pltpu.SemaphoreType.DMA for _ in range(4)]
        ),
        compiler_params=pltpu.CompilerParams(use_tc_tiling_on_sc=False),
    )
    def body(v_hbm, src_hbm, dst_hbm, out0_hbm, out1_hbm, src_v, dst_v, *rest):
        bufs, acc, sems = rest[:8], rest[8], rest[9:]
        buf_a, buf_b = bufs[:4], bufs[4:]
        sga, ssa, sgb, ssb = sems
        c = lax.axis_index("c")
        s = lax.axis_index("s")
        wid = s * _NC + c

        pltpu.sync_copy(v_hbm.at[pl.ds(s * rb, rb)], acc.at[pl.ds(s * rb, rb)])

        @pl.when(s == _NS - 1)
        def _():
            pltpu.sync_copy(v_hbm.at[pl.ds(rb * _NS, tail)],
                            acc.at[pl.ds(rb * _NS, tail)])

        plsc.subcore_barrier()

        pltpu.sync_copy(src_hbm.at[wid], src_v)
        pltpu.sync_copy(dst_hbm.at[wid], dst_v)

        def gath(j, buf, sem):
            pltpu.async_copy(v_hbm.at[src_v.at[j]], buf, sem)

        def gwait(buf, sem):
            pltpu.make_async_copy(v_hbm.at[src_v.at[0]], buf, sem).wait()

        def scat(j, buf, sem):
            pltpu.async_copy(buf, acc.at[dst_v.at[j]], sem, add=True)

        def swait(buf, sem):
            pltpu.make_async_copy(buf, acc.at[dst_v.at[0]], sem).wait()

        nouter = cpw // 8
        assert nouter >= 1
        for b in range(4):
            gath(b, buf_a[b], sga)

        def pipe(i, carry):
            @pl.when(i > 0)
            def _():
                for b in range(4):
                    swait(buf_b[b], ssb)

            for b in range(4):
                gath(8 * i + 4 + b, buf_b[b], sgb)
            for b in range(4):
                gwait(buf_a[b], sga)
            for b in range(4):
                scat(8 * i + b, buf_a[b], ssa)
            @pl.when(i < nouter - 1)
            def _():
                for b in range(4):
                    swait(buf_a[b], ssa)
                for b in range(4):
                    gath(8 * (i + 1) + b, buf_a[b], sga)

            for b in range(4):
                gwait(buf_b[b], sgb)
            for b in range(4):
                scat(8 * i + 4 + b, buf_b[b], ssb)
            return carry

        lax.fori_loop(0, nouter, pipe, 0)
        for b in range(4):
            swait(buf_a[b], ssa)
        for b in range(4):
            swait(buf_b[b], ssb)

        for r in range(8 * nouter, cpw):
            gath(r, buf_a[0], sga)
            gwait(buf_a[0], sga)
            scat(r, buf_a[0], ssa)
            swait(buf_a[0], ssa)
        plsc.subcore_barrier()

        @pl.when(c == 0)
        def _():
            pltpu.sync_copy(acc.at[pl.ds(s * rb, rb)],
                            out0_hbm.at[pl.ds(s * rb, rb)])

            @pl.when(s == _NS - 1)
            def _():
                pltpu.sync_copy(acc.at[pl.ds(rb * _NS, tail)],
                                out0_hbm.at[pl.ds(rb * _NS, tail)])

        @pl.when(c == 1)
        def _():
            pltpu.sync_copy(acc.at[pl.ds(s * rb, rb)],
                            out1_hbm.at[pl.ds(s * rb, rb)])

            @pl.when(s == _NS - 1)
            def _():
                pltpu.sync_copy(acc.at[pl.ds(rb * _NS, tail)],
                                out1_hbm.at[pl.ds(rb * _NS, tail)])

    return body(v, src2, dst2)


def _row_specs(br, dims):
    return [pl.BlockSpec((br, d), lambda i: (0, 0) if fixed else (i, 0))
            for d, fixed in dims]


def _full_spec(shape):
    nd = len(shape)
    return pl.BlockSpec(shape, lambda i: (0,) * nd)


def _blk_spec(br, d):
    return pl.BlockSpec((br, d), lambda i: (i, 0))


def _mm1_body(x_ref, w_ref, o_ref):
    o_ref[...] = jnp.dot(x_ref[...], w_ref[...], preferred_element_type=F32)


def _mm1(x, w, br):
    n = x.shape[0]
    return pl.pallas_call(
        _mm1_body,
        grid=(n // br,),
        in_specs=[_blk_spec(br, x.shape[1]), _full_spec(w.shape)],
        out_specs=_blk_spec(br, w.shape[1]),
        out_shape=jax.ShapeDtypeStruct((n, w.shape[1]), F32),
    )(x, w)


def _post_body(v_ref, a0_ref, a1_ref, b1_ref, w2_ref, b2_ref, wn_ref, o_ref):
    z = jnp.maximum(a0_ref[...] + a1_ref[...] - v_ref[...] + b1_ref[...], 0.0)
    xo = jnp.dot(z, w2_ref[...], preferred_element_type=F32) + b2_ref[...]
    o_ref[...] = jnp.dot(xo, wn_ref[...], preferred_element_type=F32)


def _post(v, a0, a1, b1, w2, b2, wnext, br):
    n, h = v.shape
    return pl.pallas_call(
        _post_body,
        grid=(n // br,),
        in_specs=[_blk_spec(br, h), _blk_spec(br, h), _blk_spec(br, h),
                  _full_spec((1, h)), _full_spec(w2.shape),
                  _full_spec((1, w2.shape[1])), _full_spec(wnext.shape)],
        out_specs=_blk_spec(br, wnext.shape[1]),
        out_shape=jax.ShapeDtypeStruct((n, wnext.shape[1]), F32),
    )(v, a0, a1, b1.reshape(1, -1), w2, b2.reshape(1, -1), wnext)


def _post2_body(v_ref, a0_ref, a1_ref, b1_ref, w2_ref, b2_ref,
                wa_ref, side_ref, wb_ref, x2_ref, v_next_ref):
    z = jnp.maximum(a0_ref[...] + a1_ref[...] - v_ref[...] + b1_ref[...], 0.0)
    xo = jnp.dot(z, w2_ref[...], preferred_element_type=F32) + b2_ref[...]
    x2_ref[...] = xo
    v_next_ref[...] = (jnp.dot(side_ref[...], wa_ref[...],
                               preferred_element_type=F32)
                       + jnp.dot(xo, wb_ref[...], preferred_element_type=F32))


def _post2(v, a0, a1, b1, w2, b2, wa, side, wb, br):
    n, h = v.shape
    ho = w2.shape[1]
    hn = wb.shape[1]
    return pl.pallas_call(
        _post2_body,
        grid=(n // br,),
        in_specs=[_blk_spec(br, h), _blk_spec(br, h), _blk_spec(br, h),
                  _full_spec((1, h)), _full_spec(w2.shape),
                  _full_spec((1, ho)), _full_spec(wa.shape),
                  _blk_spec(br, side.shape[1]), _full_spec(wb.shape)],
        out_specs=(_blk_spec(br, ho), _blk_spec(br, hn)),
        out_shape=(jax.ShapeDtypeStruct((n, ho), F32),
                   jax.ShapeDtypeStruct((n, hn), F32)),
    )(v, a0, a1, b1.reshape(1, -1), w2, b2.reshape(1, -1), wa, side, wb)


def _pool_body(v_ref, a0_ref, a1_ref, b1_ref, w2_ref, b2_ref, bat_ref,
               o_ref, cnt_ref, *, g, br, nblk):
    i = pl.program_id(0)

    @pl.when(i == 0)
    def _():
        o_ref[...] = jnp.zeros_like(o_ref)
        cnt_ref[...] = jnp.zeros_like(cnt_ref)

    z = jnp.maximum(a0_ref[...] + a1_ref[...] - v_ref[...] + b1_ref[...], 0.0)
    xo = jnp.dot(z, w2_ref[...], preferred_element_type=F32) + b2_ref[...]
    c = xo.shape[1]
    gid = lax.broadcasted_iota(jnp.int32, (g, br), 0).astype(F32)
    oh = jnp.where(jnp.broadcast_to(bat_ref[0], (g, br)) == gid, 1.0, 0.0)
    o_ref[...] += jnp.dot(oh, xo, preferred_element_type=F32)
    cnt_ref[...] += jnp.dot(oh, jnp.ones((br, c), F32),
                            preferred_element_type=F32)

    @pl.when(i == nblk - 1)
    def _():
        o_ref[...] = o_ref[...] / jnp.maximum(cnt_ref[...], 1.0)


def _pool(v, a0, a1, b1, w2, b2, batchf, g, br):
    n, h = v.shape
    c = w2.shape[1]
    nblk = n // br
    return pl.pallas_call(
        functools.partial(_pool_body, g=g, br=br, nblk=nblk),
        grid=(nblk,),
        in_specs=[_blk_spec(br, h), _blk_spec(br, h), _blk_spec(br, h),
                  _full_spec((1, h)), _full_spec(w2.shape),
                  _full_spec((1, c)),
                  pl.BlockSpec((1, 1, br), lambda i: (i, 0, 0))],
        out_specs=pl.BlockSpec((g, c), lambda i: (0, 0)),
        out_shape=jax.ShapeDtypeStruct((g, c), F32),
        scratch_shapes=[pltpu.VMEM((g, c), F32)],
    )(v, a0, a1, b1.reshape(1, -1), w2, b2.reshape(1, -1), batchf)


def kernel(x, edge_index, batch,
           W11, b11, W12, b12,
           W21, b21, W22, b22,
           W31, b31, W32, b32,
           W41, b41, W42, b42):
    n, d = x.shape
    h = W11.shape[1]
    e = edge_index.shape[1]
    g = 64
    c_out = W42.shape[1]
    br = 2000

    src2 = edge_index[0].reshape(_NW, e // (_NW * _K), _K)
    dst2 = edge_index[1].reshape(_NW, e // (_NW * _K), _K)
    batchf = batch.astype(F32).reshape(n // br, 1, br)

    W31a, W31b = W31[:d], W31[d:]
    W41a, W41b = W41[:h], W41[h:]

    v1 = _mm1(x, W11, br)
    a10, a11 = _agg_sc(v1, src2, dst2, n, h)
    v2 = _post(v1, a10, a11, b11, W12, b12, W21, br)
    a20, a21 = _agg_sc(v2, src2, dst2, n, h)
    x2, v3 = _post2(v2, a20, a21, b21, W22, b22, W31a, x, W31b, br)
    a30, a31 = _agg_sc(v3, src2, dst2, n, h)
    _, v4 = _post2(v3, a30, a31, b31, W32, b32, W41b, x2, W41a, br)
    a40, a41 = _agg_sc(v4, src2, dst2, n, h)
    return _pool(v4, a40, a41, b41, W42, b42, batchf, g, br)


# speedup vs baseline: 18.5140x; 1.1156x over previous
import functools

import jax
import jax.numpy as jnp
from jax import lax
from jax.experimental import pallas as pl
from jax.experimental.pallas import tpu as pltpu
from jax.experimental.pallas import tpu_sc as plsc

F32 = jnp.float32

_NC = 2
_NS = 16
_NW = _NC * _NS

_K = 80


def _agg_sc(v, src2, dst2, n_nodes, h):
    cpw = src2.shape[1]
    rb = (n_nodes // _NS) // 8 * 8
    tail = n_nodes - rb * _NS

    mesh = plsc.VectorSubcoreMesh(core_axis_name="c", subcore_axis_name="s")

    @functools.partial(
        pl.kernel,
        mesh=mesh,
        out_type=jax.ShapeDtypeStruct((n_nodes, 2 * h), F32),
        scratch_types=(
            [pltpu.VMEM((cpw, _K), jnp.int32),
             pltpu.VMEM((cpw, _K), jnp.int32)]
            + [pltpu.VMEM((_K, h), F32) for _ in range(8)]
            + [pltpu.VMEM_SHARED((n_nodes, h), F32)]
            + [pltpu.SemaphoreType.DMA for _ in range(4)]
        ),
        compiler_params=pltpu.CompilerParams(use_tc_tiling_on_sc=False),
    )
    def body(v_hbm, src_hbm, dst_hbm, out_hbm, src_v, dst_v, *rest):
        bufs, acc, sems = rest[:8], rest[8], rest[9:]
        buf_a, buf_b = bufs[:4], bufs[4:]
        sga, ssa, sgb, ssb = sems
        c = lax.axis_index("c")
        s = lax.axis_index("s")
        wid = s * _NC + c

        col = c * h
        pltpu.sync_copy(v_hbm.at[pl.ds(s * rb, rb)], acc.at[pl.ds(s * rb, rb)])

        @pl.when(s == _NS - 1)
        def _():
            pltpu.sync_copy(v_hbm.at[pl.ds(rb * _NS, tail)],
                            acc.at[pl.ds(rb * _NS, tail)])

        plsc.subcore_barrier()

        pltpu.sync_copy(src_hbm.at[wid], src_v)
        pltpu.sync_copy(dst_hbm.at[wid], dst_v)

        def gath(j, buf, sem):
            pltpu.async_copy(v_hbm.at[src_v.at[j]], buf, sem)

        def gwait(buf, sem):
            pltpu.make_async_copy(v_hbm.at[src_v.at[0]], buf, sem).wait()

        def scat(j, buf, sem):
            pltpu.async_copy(buf, acc.at[dst_v.at[j]], sem, add=True)

        def swait(buf, sem):
            pltpu.make_async_copy(buf, acc.at[dst_v.at[0]], sem).wait()

        nouter = cpw // 8
        assert nouter >= 1
        for b in range(4):
            gath(b, buf_a[b], sga)

        def pipe(i, carry):
            @pl.when(i > 0)
            def _():
                for b in range(4):
                    swait(buf_b[b], ssb)

            for b in range(4):
                gath(8 * i + 4 + b, buf_b[b], sgb)
            for b in range(4):
                gwait(buf_a[b], sga)
            for b in range(4):
                scat(8 * i + b, buf_a[b], ssa)
            @pl.when(i < nouter - 1)
            def _():
                for b in range(4):
                    swait(buf_a[b], ssa)
                for b in range(4):
                    gath(8 * (i + 1) + b, buf_a[b], sga)

            for b in range(4):
                gwait(buf_b[b], sgb)
            for b in range(4):
                scat(8 * i + 4 + b, buf_b[b], ssb)
            return carry

        lax.fori_loop(0, nouter, pipe, 0)
        for b in range(4):
            swait(buf_a[b], ssa)
        for b in range(4):
            swait(buf_b[b], ssb)

        for r in range(8 * nouter, cpw):
            gath(r, buf_a[0], sga)
            gwait(buf_a[0], sga)
            scat(r, buf_a[0], ssa)
            swait(buf_a[0], ssa)
        plsc.subcore_barrier()

        pltpu.sync_copy(acc.at[pl.ds(s * rb, rb)],
                        out_hbm.at[pl.ds(s * rb, rb), pl.ds(col, h)])

        @pl.when(s == _NS - 1)
        def _():
            pltpu.sync_copy(acc.at[pl.ds(rb * _NS, tail)],
                            out_hbm.at[pl.ds(rb * _NS, tail), pl.ds(col, h)])

    return body(v, src2, dst2)


def _full_spec(shape):
    nd = len(shape)
    return pl.BlockSpec(shape, lambda i: (0,) * nd)


def _blk_spec(br, d):
    return pl.BlockSpec((br, d), lambda i: (i, 0))


def _mm1_body(x_ref, w_ref, o_ref):
    o_ref[...] = jnp.dot(x_ref[...], w_ref[...], preferred_element_type=F32)


def _mm1(x, w, br):
    n = x.shape[0]
    return pl.pallas_call(
        _mm1_body,
        grid=(n // br,),
        in_specs=[_blk_spec(br, x.shape[1]), _full_spec(w.shape)],
        out_specs=_blk_spec(br, w.shape[1]),
        out_shape=jax.ShapeDtypeStruct((n, w.shape[1]), F32),
    )(x, w)


def _relu_sum(a_ref, v_ref, b1_ref, h):
    return jnp.maximum(a_ref[:, :h] + a_ref[:, h:] - v_ref[...] + b1_ref[...],
                       0.0)


def _post_body(a_ref, v_ref, b1_ref, w2_ref, b2_ref, wn_ref, o_ref):
    h = w2_ref.shape[0]
    z = _relu_sum(a_ref, v_ref, b1_ref, h)
    xo = jnp.dot(z, w2_ref[...], preferred_element_type=F32) + b2_ref[...]
    o_ref[...] = jnp.dot(xo, wn_ref[...], preferred_element_type=F32)


def _post(a, v, b1, w2, b2, wnext, br):
    n = a.shape[0]
    h = w2.shape[0]
    hn = wnext.shape[1]
    return pl.pallas_call(
        _post_body,
        grid=(n // br,),
        in_specs=[_blk_spec(br, 2 * h), _blk_spec(br, h), _full_spec((1, h)),
                  _full_spec(w2.shape), _full_spec((1, w2.shape[1])),
                  _full_spec(wnext.shape)],
        out_specs=_blk_spec(br, hn),
        out_shape=jax.ShapeDtypeStruct((n, hn), F32),
    )(a, v, b1.reshape(1, -1), w2, b2.reshape(1, -1), wnext)


def _post2_body(a_ref, v_ref, b1_ref, w2_ref, b2_ref,
                wa_ref, side_ref, wb_ref, x2_ref, v_next_ref):
    h = w2_ref.shape[0]
    z = _relu_sum(a_ref, v_ref, b1_ref, h)
    xo = jnp.dot(z, w2_ref[...], preferred_element_type=F32) + b2_ref[...]
    x2_ref[...] = xo
    v_next_ref[...] = (
        jnp.dot(side_ref[...], wa_ref[...], preferred_element_type=F32)
        + jnp.dot(xo, wb_ref[...], preferred_element_type=F32))


def _post2(a, v, b1, w2, b2, wa, side, wb, br):
    n = a.shape[0]
    h = w2.shape[0]
    ho = w2.shape[1]
    hn = wb.shape[1]
    return pl.pallas_call(
        _post2_body,
        grid=(n // br,),
        in_specs=[_blk_spec(br, 2 * h), _blk_spec(br, h), _full_spec((1, h)),
                  _full_spec(w2.shape), _full_spec((1, ho)),
                  _full_spec(wa.shape), _blk_spec(br, side.shape[1]),
                  _full_spec(wb.shape)],
        out_specs=(_blk_spec(br, ho), _blk_spec(br, hn)),
        out_shape=(jax.ShapeDtypeStruct((n, ho), F32),
                   jax.ShapeDtypeStruct((n, hn), F32)),
    )(a, v, b1.reshape(1, -1), w2, b2.reshape(1, -1), wa, side, wb)


def _pool_body(a_ref, v_ref, b1_ref, w2_ref, b2_ref, bat_ref,
               o_ref, cnt_ref, *, g, br, nblk):
    i = pl.program_id(0)

    @pl.when(i == 0)
    def _():
        o_ref[...] = jnp.zeros_like(o_ref)
        cnt_ref[...] = jnp.zeros_like(cnt_ref)

    h = w2_ref.shape[0]
    z = _relu_sum(a_ref, v_ref, b1_ref, h)
    xo = jnp.dot(z, w2_ref[...], preferred_element_type=F32) + b2_ref[...]
    c = xo.shape[1]
    gid = lax.broadcasted_iota(jnp.int32, (g, br), 0).astype(F32)
    oh = jnp.where(jnp.broadcast_to(bat_ref[0], (g, br)) == gid, 1.0, 0.0)
    o_ref[...] += jnp.dot(oh, xo, preferred_element_type=F32)
    cnt_ref[...] += jnp.dot(oh, jnp.ones((br, c), F32),
                            preferred_element_type=F32)

    @pl.when(i == nblk - 1)
    def _():
        o_ref[...] = o_ref[...] / jnp.maximum(cnt_ref[...], 1.0)


def _pool(a, v, b1, w2, b2, batchf, g, br):
    n = a.shape[0]
    h = w2.shape[0]
    c = w2.shape[1]
    nblk = n // br
    return pl.pallas_call(
        functools.partial(_pool_body, g=g, br=br, nblk=nblk),
        grid=(nblk,),
        in_specs=[_blk_spec(br, 2 * h), _blk_spec(br, h), _full_spec((1, h)),
                  _full_spec(w2.shape), _full_spec((1, c)),
                  pl.BlockSpec((1, 1, br), lambda i: (i, 0, 0))],
        out_specs=pl.BlockSpec((g, c), lambda i: (0, 0)),
        out_shape=jax.ShapeDtypeStruct((g, c), F32),
        scratch_shapes=[pltpu.VMEM((g, c), F32)],
    )(a, v, b1.reshape(1, -1), w2, b2.reshape(1, -1), batchf)


def kernel(x, edge_index, batch,
           W11, b11, W12, b12,
           W21, b21, W22, b22,
           W31, b31, W32, b32,
           W41, b41, W42, b42):
    n, d = x.shape
    h = W11.shape[1]
    e = edge_index.shape[1]
    g = 64
    br = 2000

    src2 = edge_index[0].reshape(_NW, e // (_NW * _K), _K)
    dst2 = edge_index[1].reshape(_NW, e // (_NW * _K), _K)
    batchf = batch.astype(F32).reshape(n // br, 1, br)

    W31a, W31b = W31[:d], W31[d:]
    W41a, W41b = W41[:h], W41[h:]

    v1 = _mm1(x, W11, br)
    a1 = _agg_sc(v1, src2, dst2, n, h)
    v2 = _post(a1, v1, b11, W12, b12, W21, br)
    a2 = _agg_sc(v2, src2, dst2, n, h)
    x2, v3 = _post2(a2, v2, b21, W22, b22, W31a, x, W31b, br)
    a3 = _agg_sc(v3, src2, dst2, n, h)
    _, v4 = _post2(a3, v3, b31, W32, b32, W41b, x2, W41a, br)
    a4 = _agg_sc(v4, src2, dst2, n, h)
    return _pool(a4, v4, b41, W42, b42, batchf, g, br)


# speedup vs baseline: 19.0401x; 1.0284x over previous
import functools

import jax
import jax.numpy as jnp
from jax import lax
from jax.experimental import pallas as pl
from jax.experimental.pallas import tpu as pltpu
from jax.experimental.pallas import tpu_sc as plsc

F32 = jnp.float32

_NC = 2
_NS = 16
_NW = _NC * _NS

_K = 128


def _agg_sc(v, er, n_nodes, h):
    nchunk = er.shape[1]
    base_c = nchunk // _NW
    xtra = nchunk - base_c * _NW
    cmax = base_c + (1 if xtra else 0)
    rb = (n_nodes // _NS) // 8 * 8
    tail = n_nodes - rb * _NS

    mesh = plsc.VectorSubcoreMesh(core_axis_name="c", subcore_axis_name="s")

    @functools.partial(
        pl.kernel,
        mesh=mesh,
        out_type=jax.ShapeDtypeStruct((n_nodes, 2 * h), F32),
        scratch_types=(
            [pltpu.VMEM((cmax, _K), jnp.int32),
             pltpu.VMEM((cmax, _K), jnp.int32)]
            + [pltpu.VMEM((_K, h), F32) for _ in range(8)]
            + [pltpu.VMEM_SHARED((n_nodes, h), F32)]
            + [pltpu.SemaphoreType.DMA for _ in range(5)]
        ),
        compiler_params=pltpu.CompilerParams(use_tc_tiling_on_sc=False),
    )
    def body(v_hbm, er_hbm, out_hbm, src_v, dst_v, *rest):
        bufs, acc, sems = rest[:8], rest[8], rest[9:]
        buf_a, buf_b = bufs[:4], bufs[4:]
        sga, ssa, sgb, ssb, semi = sems
        c = lax.axis_index("c")
        s = lax.axis_index("s")
        wid = s * _NC + c

        cbase = wid * base_c + jnp.minimum(wid, xtra)
        ncw = base_c + jnp.where(wid < xtra, 1, 0)

        pltpu.async_copy(er_hbm.at[0, pl.ds(cbase, base_c)],
                         src_v.at[pl.ds(0, base_c)], semi)
        pltpu.async_copy(er_hbm.at[1, pl.ds(cbase, base_c)],
                         dst_v.at[pl.ds(0, base_c)], semi)
        if xtra:
            @pl.when(wid < xtra)
            def _():
                pltpu.async_copy(er_hbm.at[0, pl.ds(cbase + base_c, 1)],
                                 src_v.at[pl.ds(base_c, 1)], semi)
                pltpu.async_copy(er_hbm.at[1, pl.ds(cbase + base_c, 1)],
                                 dst_v.at[pl.ds(base_c, 1)], semi)

        col = c * h
        pltpu.sync_copy(v_hbm.at[pl.ds(s * rb, rb)], acc.at[pl.ds(s * rb, rb)])

        @pl.when(s == _NS - 1)
        def _():
            pltpu.sync_copy(v_hbm.at[pl.ds(rb * _NS, tail)],
                            acc.at[pl.ds(rb * _NS, tail)])

        plsc.subcore_barrier()

        pltpu.make_async_copy(er_hbm.at[0, pl.ds(0, base_c)],
                              src_v.at[pl.ds(0, base_c)], semi).wait()
        pltpu.make_async_copy(er_hbm.at[0, pl.ds(0, base_c)],
                              dst_v.at[pl.ds(0, base_c)], semi).wait()
        if xtra:
            @pl.when(wid < xtra)
            def _():
                pltpu.make_async_copy(er_hbm.at[0, pl.ds(0, 1)],
                                      src_v.at[pl.ds(0, 1)], semi).wait()
                pltpu.make_async_copy(er_hbm.at[0, pl.ds(0, 1)],
                                      dst_v.at[pl.ds(0, 1)], semi).wait()

        def gath(j, buf, sem):
            pltpu.async_copy(v_hbm.at[src_v.at[j]], buf, sem)

        def gwait(buf, sem):
            pltpu.make_async_copy(v_hbm.at[src_v.at[0]], buf, sem).wait()

        def scat(j, buf, sem):
            pltpu.async_copy(buf, acc.at[dst_v.at[j]], sem, add=True)

        def swait(buf, sem):
            pltpu.make_async_copy(buf, acc.at[dst_v.at[0]], sem).wait()

        nouter = base_c // 8
        assert nouter >= 1
        for b in range(4):
            gath(b, buf_a[b], sga)

        def pipe(i, carry):
            @pl.when(i > 0)
            def _():
                for b in range(4):
                    swait(buf_b[b], ssb)

            for b in range(4):
                gath(8 * i + 4 + b, buf_b[b], sgb)
            for b in range(4):
                gwait(buf_a[b], sga)
            for b in range(4):
                scat(8 * i + b, buf_a[b], ssa)
            @pl.when(i < nouter - 1)
            def _():
                for b in range(4):
                    swait(buf_a[b], ssa)
                for b in range(4):
                    gath(8 * (i + 1) + b, buf_a[b], sga)

            for b in range(4):
                gwait(buf_b[b], sgb)
            for b in range(4):
                scat(8 * i + 4 + b, buf_b[b], ssb)
            return carry

        lax.fori_loop(0, nouter, pipe, 0)
        for b in range(4):
            swait(buf_a[b], ssa)
        for b in range(4):
            swait(buf_b[b], ssb)

        def tail_chunk(r, carry):
            gath(r, buf_a[0], sga)
            gwait(buf_a[0], sga)
            scat(r, buf_a[0], ssa)
            swait(buf_a[0], ssa)
            return carry

        lax.fori_loop(8 * nouter, ncw, tail_chunk, 0)
        plsc.subcore_barrier()

        pltpu.sync_copy(acc.at[pl.ds(s * rb, rb)],
                        out_hbm.at[pl.ds(s * rb, rb), pl.ds(col, h)])

        @pl.when(s == _NS - 1)
        def _():
            pltpu.sync_copy(acc.at[pl.ds(rb * _NS, tail)],
                            out_hbm.at[pl.ds(rb * _NS, tail), pl.ds(col, h)])

    return body(v, er)


def _full_spec(shape):
    nd = len(shape)
    return pl.BlockSpec(shape, lambda i: (0,) * nd)


def _blk_spec(br, d):
    return pl.BlockSpec((br, d), lambda i: (i, 0))


def _mm1_body(x_ref, w_ref, o_ref):
    o_ref[...] = jnp.dot(x_ref[...], w_ref[...], preferred_element_type=F32)


def _mm1(x, w, br):
    n = x.shape[0]
    return pl.pallas_call(
        _mm1_body,
        grid=(n // br,),
        in_specs=[_blk_spec(br, x.shape[1]), _full_spec(w.shape)],
        out_specs=_blk_spec(br, w.shape[1]),
        out_shape=jax.ShapeDtypeStruct((n, w.shape[1]), F32),
    )(x, w)


def _relu_sum(a_ref, v_ref, b1_ref, h):
    return jnp.maximum(a_ref[:, :h] + a_ref[:, h:] - v_ref[...] + b1_ref[...],
                       0.0)


def _post_body(a_ref, v_ref, b1_ref, w2_ref, b2_ref, wn_ref, o_ref):
    h = w2_ref.shape[0]
    z = _relu_sum(a_ref, v_ref, b1_ref, h)
    xo = jnp.dot(z, w2_ref[...], preferred_element_type=F32) + b2_ref[...]
    o_ref[...] = jnp.dot(xo, wn_ref[...], preferred_element_type=F32)


def _post(a, v, b1, w2, b2, wnext, br):
    n = a.shape[0]
    h = w2.shape[0]
    hn = wnext.shape[1]
    return pl.pallas_call(
        _post_body,
        grid=(n // br,),
        in_specs=[_blk_spec(br, 2 * h), _blk_spec(br, h), _full_spec((1, h)),
                  _full_spec(w2.shape), _full_spec((1, w2.shape[1])),
                  _full_spec(wnext.shape)],
        out_specs=_blk_spec(br, hn),
        out_shape=jax.ShapeDtypeStruct((n, hn), F32),
    )(a, v, b1.reshape(1, -1), w2, b2.reshape(1, -1), wnext)


def _post2_body(a_ref, v_ref, b1_ref, w2_ref, b2_ref,
                wa_ref, side_ref, wb_ref, x2_ref, v_next_ref):
    h = w2_ref.shape[0]
    z = _relu_sum(a_ref, v_ref, b1_ref, h)
    xo = jnp.dot(z, w2_ref[...], preferred_element_type=F32) + b2_ref[...]
    x2_ref[...] = xo
    v_next_ref[...] = (
        jnp.dot(side_ref[...], wa_ref[...], preferred_element_type=F32)
        + jnp.dot(xo, wb_ref[...], preferred_element_type=F32))


def _post2(a, v, b1, w2, b2, wa, side, wb, br):
    n = a.shape[0]
    h = w2.shape[0]
    ho = w2.shape[1]
    hn = wb.shape[1]
    return pl.pallas_call(
        _post2_body,
        grid=(n // br,),
        in_specs=[_blk_spec(br, 2 * h), _blk_spec(br, h), _full_spec((1, h)),
                  _full_spec(w2.shape), _full_spec((1, ho)),
                  _full_spec(wa.shape), _blk_spec(br, side.shape[1]),
                  _full_spec(wb.shape)],
        out_specs=(_blk_spec(br, ho), _blk_spec(br, hn)),
        out_shape=(jax.ShapeDtypeStruct((n, ho), F32),
                   jax.ShapeDtypeStruct((n, hn), F32)),
    )(a, v, b1.reshape(1, -1), w2, b2.reshape(1, -1), wa, side, wb)


def _pool_body(a_ref, v_ref, b1_ref, w2_ref, b2_ref, bat_ref,
               o_ref, cnt_ref, *, g, br, nblk):
    i = pl.program_id(0)

    @pl.when(i == 0)
    def _():
        o_ref[...] = jnp.zeros_like(o_ref)
        cnt_ref[...] = jnp.zeros_like(cnt_ref)

    h = w2_ref.shape[0]
    z = _relu_sum(a_ref, v_ref, b1_ref, h)
    xo = jnp.dot(z, w2_ref[...], preferred_element_type=F32) + b2_ref[...]
    c = xo.shape[1]
    gid = lax.broadcasted_iota(jnp.int32, (g, br), 0).astype(F32)
    oh = jnp.where(jnp.broadcast_to(bat_ref[0], (g, br)) == gid, 1.0, 0.0)
    o_ref[...] += jnp.dot(oh, xo, preferred_element_type=F32)
    cnt_ref[...] += jnp.dot(oh, jnp.ones((br, c), F32),
                            preferred_element_type=F32)

    @pl.when(i == nblk - 1)
    def _():
        o_ref[...] = o_ref[...] / jnp.maximum(cnt_ref[...], 1.0)


def _pool(a, v, b1, w2, b2, batchf, g, br):
    n = a.shape[0]
    h = w2.shape[0]
    c = w2.shape[1]
    nblk = n // br
    return pl.pallas_call(
        functools.partial(_pool_body, g=g, br=br, nblk=nblk),
        grid=(nblk,),
        in_specs=[_blk_spec(br, 2 * h), _blk_spec(br, h), _full_spec((1, h)),
                  _full_spec(w2.shape), _full_spec((1, c)),
                  pl.BlockSpec((1, 1, br), lambda i: (i, 0, 0))],
        out_specs=pl.BlockSpec((g, c), lambda i: (0, 0)),
        out_shape=jax.ShapeDtypeStruct((g, c), F32),
        scratch_shapes=[pltpu.VMEM((g, c), F32)],
    )(a, v, b1.reshape(1, -1), w2, b2.reshape(1, -1), batchf)


def kernel(x, edge_index, batch,
           W11, b11, W12, b12,
           W21, b21, W22, b22,
           W31, b31, W32, b32,
           W41, b41, W42, b42):
    n, d = x.shape
    h = W11.shape[1]
    e = edge_index.shape[1]
    g = 64
    br = 2000

    er = edge_index.reshape(2, e // _K, _K)
    batchf = batch.astype(F32).reshape(n // br, 1, br)

    W31a, W31b = W31[:d], W31[d:]
    W41a, W41b = W41[:h], W41[h:]

    v1 = _mm1(x, W11, br)
    a1 = _agg_sc(v1, er, n, h)
    v2 = _post(a1, v1, b11, W12, b12, W21, br)
    a2 = _agg_sc(v2, er, n, h)
    x2, v3 = _post2(a2, v2, b21, W22, b22, W31a, x, W31b, br)
    a3 = _agg_sc(v3, er, n, h)
    _, v4 = _post2(a3, v3, b31, W32, b32, W41b, x2, W41a, br)
    a4 = _agg_sc(v4, er, n, h)
    return _pool(a4, v4, b41, W42, b42, batchf, g, br)
